# Initial kernel scaffold; baseline (speedup 1.0000x reference)
import jax
import jax.numpy as jnp
from jax.experimental import pallas as pl


def kernel(features, W1, b1, W_obj, b_obj, W_tr, b_tr):
    raise NotImplementedError("write your pallas kernel here")


# speedup vs baseline: 1.1321x; 1.1321x over previous
import jax
import jax.numpy as jnp
from jax.experimental import pallas as pl
from jax.experimental.pallas import tpu as pltpu

_B, _C, _H, _W = 2, 96, 128, 128
_MID = 512
_NOUT = 18 + 36
_ROWS = 8


def _rpn_body(x_ref, w1_ref, b1_ref, wc_ref, bc_ref, out_ref):
    i = pl.program_id(1)
    acc = jnp.zeros((_ROWS * _W, _MID), jnp.float32)
    for di in range(3):
        for dj in range(3):
            lhs = x_ref[0, pl.ds(i * _ROWS + di, _ROWS), pl.ds(dj, _W), :]
            acc += jnp.dot(
                lhs.reshape(_ROWS * _W, _C),
                w1_ref[di * 3 + dj],
                preferred_element_type=jnp.float32,
            )
    h = jnp.maximum(acc + b1_ref[0], 0.0)
    out = jnp.dot(h, wc_ref[...], preferred_element_type=jnp.float32) + bc_ref[0]
    out_ref[...] = out.reshape(1, _ROWS, _W, _NOUT)


def kernel(features, W1, b1, W_obj, b_obj, W_tr, b_tr):
    xp = jnp.pad(
        jnp.transpose(features, (0, 2, 3, 1)),
        ((0, 0), (1, 1), (1, 1), (0, 0)),
    )
    w1m = jnp.transpose(W1, (2, 3, 1, 0)).reshape(9, _C, _MID)
    wc = jnp.concatenate(
        [W_obj.reshape(18, _MID).T, W_tr.reshape(36, _MID).T], axis=1
    )
    bc = jnp.concatenate([b_obj, b_tr]).reshape(1, _NOUT)
    b1m = b1.reshape(1, _MID)

    out = pl.pallas_call(
        _rpn_body,
        grid=(_B, _H // _ROWS),
        in_specs=[
            pl.BlockSpec((1, _H + 2, _W + 2, _C), lambda b, i: (b, 0, 0, 0)),
            pl.BlockSpec((9, _C, _MID), lambda b, i: (0, 0, 0)),
            pl.BlockSpec((1, _MID), lambda b, i: (0, 0)),
            pl.BlockSpec((_MID, _NOUT), lambda b, i: (0, 0)),
            pl.BlockSpec((1, _NOUT), lambda b, i: (0, 0)),
        ],
        out_specs=pl.BlockSpec((1, _ROWS, _W, _NOUT), lambda b, i: (b, i, 0, 0)),
        out_shape=jax.ShapeDtypeStruct((_B, _H, _W, _NOUT), jnp.float32),
        compiler_params=pltpu.CompilerParams(
            dimension_semantics=("parallel", "arbitrary"),
        ),
    )(xp, w1m, b1m, wc, bc)

    obj = out[..., :18].reshape(_B, -1, 2)
    tr = out[..., 18:].reshape(_B, -1, 4)
    return (obj, tr)


# speedup vs baseline: 1.1382x; 1.0054x over previous
import jax
import jax.numpy as jnp
from jax.experimental import pallas as pl
from jax.experimental.pallas import tpu as pltpu

_B, _C, _H, _W = 2, 96, 128, 128
_MID = 512
_NOUT = 18 + 36
_ROWS = 8


def _rpn_body(x_ref, w1_ref, b1_ref, wc_ref, bc_ref, out_ref):
    i = pl.program_id(1)
    acc = jnp.zeros((_ROWS * _W, _MID), jnp.float32)
    for di in range(3):
        for dj in range(3):
            lhs = x_ref[0, pl.ds(i * _ROWS + di, _ROWS), pl.ds(dj, _W), :]
            acc += jnp.dot(
                lhs.reshape(_ROWS * _W, _C),
                w1_ref[di * 3 + dj],
                preferred_element_type=jnp.float32,
            )
    h = jnp.maximum(acc + b1_ref[0], 0.0).astype(jnp.bfloat16)
    out = jnp.dot(h, wc_ref[...], preferred_element_type=jnp.float32) + bc_ref[0]
    out_ref[...] = out.reshape(1, _ROWS, _W, _NOUT)


def kernel(features, W1, b1, W_obj, b_obj, W_tr, b_tr):
    xp = jnp.pad(
        jnp.transpose(features, (0, 2, 3, 1)),
        ((0, 0), (1, 1), (1, 1), (0, 0)),
    ).astype(jnp.bfloat16)
    w1m = jnp.transpose(W1, (2, 3, 1, 0)).reshape(9, _C, _MID).astype(jnp.bfloat16)
    wc = jnp.concatenate(
        [W_obj.reshape(18, _MID).T, W_tr.reshape(36, _MID).T], axis=1
    ).astype(jnp.bfloat16)
    bc = jnp.concatenate([b_obj, b_tr]).reshape(1, _NOUT)
    b1m = b1.reshape(1, _MID)

    out = pl.pallas_call(
        _rpn_body,
        grid=(_B, _H // _ROWS),
        in_specs=[
            pl.BlockSpec((1, _H + 2, _W + 2, _C), lambda b, i: (b, 0, 0, 0)),
            pl.BlockSpec((9, _C, _MID), lambda b, i: (0, 0, 0)),
            pl.BlockSpec((1, _MID), lambda b, i: (0, 0)),
            pl.BlockSpec((_MID, _NOUT), lambda b, i: (0, 0)),
            pl.BlockSpec((1, _NOUT), lambda b, i: (0, 0)),
        ],
        out_specs=pl.BlockSpec((1, _ROWS, _W, _NOUT), lambda b, i: (b, i, 0, 0)),
        out_shape=jax.ShapeDtypeStruct((_B, _H, _W, _NOUT), jnp.float32),
        compiler_params=pltpu.CompilerParams(
            dimension_semantics=("parallel", "arbitrary"),
        ),
    )(xp, w1m, b1m, wc, bc)

    obj = out[..., :18].reshape(_B, -1, 2)
    tr = out[..., 18:].reshape(_B, -1, 4)
    return (obj, tr)
